# baseline (speedup 1.0000x reference)
import jax
import jax.numpy as jnp
from jax.experimental import pallas as pl
from jax.experimental.pallas import tpu as pltpu

B, T = 16, 750
DW, DL, DM = 1280, 1024, 1024
D = DW + DL + DM
CODE_DIM, K = 256, 1024
N = B * T

TILE_M = 240
NT = N // TILE_M


def _body(w_ref, l_ref, m_ref, wew_ref, wel_ref, wem_ref, be_ref, cbt_ref,
          cbb_ref, wd_ref, bd_ref,
          recon_ref, codes_ref, commit_ref, ent_ref,
          acc_ref, csum_ref):
    i = pl.program_id(0)

    @pl.when(i == 0)
    def _init():
        acc_ref[...] = jnp.zeros_like(acc_ref)
        cbt = cbt_ref[...]
        acc_ref[1:2, :] = jnp.sum(cbt * cbt, axis=0, keepdims=True)
        csum_ref[...] = jnp.zeros_like(csum_ref)

    ze = (jnp.dot(w_ref[...], wew_ref[...], preferred_element_type=jnp.float32)
          + jnp.dot(l_ref[...], wel_ref[...], preferred_element_type=jnp.float32)
          + jnp.dot(m_ref[...], wem_ref[...], preferred_element_type=jnp.float32)
          + be_ref[...])

    z2 = jnp.sum(ze * ze, axis=1, keepdims=True)
    zc = jnp.dot(ze, cbt_ref[...], preferred_element_type=jnp.float32)
    d = z2 - 2.0 * zc + acc_ref[1:2, :]

    dmin = jnp.min(d, axis=1, keepdims=True)
    kiota = jax.lax.broadcasted_iota(jnp.int32, d.shape, 1)
    codes = jnp.min(jnp.where(d == dmin, kiota, K), axis=1,
                    keepdims=True)
    codes_ref[...] = codes

    p = jnp.exp(dmin - d)
    s = jnp.sum(p, axis=1, keepdims=True)
    acc_ref[0:1, :] = acc_ref[0:1, :] + jnp.sum(p / s, axis=0, keepdims=True)
    csum_ref[0:1, 0:1] = csum_ref[0:1, 0:1] + jnp.sum(dmin, axis=0,
                                                      keepdims=True)

    onehot = (kiota == codes).astype(jnp.bfloat16)
    zq = jnp.dot(onehot, cbb_ref[...],
                 preferred_element_type=jnp.float32)
    zq_st = ze + (zq - ze)
    recon_ref[...] = (jnp.dot(zq_st.astype(jnp.bfloat16), wd_ref[...],
                              preferred_element_type=jnp.float32)
                      + bd_ref[...])

    @pl.when(i == NT - 1)
    def _fin():
        commit_ref[...] = csum_ref[0:1, 0:1] / (N * CODE_DIM)
        avg = acc_ref[0:1, :] / N
        ent_ref[...] = jnp.sum(avg * jnp.log(avg + 1e-10), axis=1,
                               keepdims=True)


@jax.jit
def kernel(whisper_feat, wavlm_feat, muq_feat, W_enc, b_enc, codebook,
           W_dec, b_dec):
    wf = whisper_feat.reshape(N, DW)
    lf = wavlm_feat.reshape(N, DL)
    mf = muq_feat.reshape(N, DM)
    wew = W_enc[:DW]
    wel = W_enc[DW:DW + DL]
    wem = W_enc[DW + DL:]
    cbt = codebook.T
    cbb = codebook.astype(jnp.bfloat16)
    wdb = W_dec.astype(jnp.bfloat16)

    recon, codes, commit, ent = pl.pallas_call(
        _body,
        grid=(NT,),
        in_specs=[
            pl.BlockSpec((TILE_M, DW), lambda i: (i, 0)),
            pl.BlockSpec((TILE_M, DL), lambda i: (i, 0)),
            pl.BlockSpec((TILE_M, DM), lambda i: (i, 0)),
            pl.BlockSpec((DW, CODE_DIM), lambda i: (0, 0)),
            pl.BlockSpec((DL, CODE_DIM), lambda i: (0, 0)),
            pl.BlockSpec((DM, CODE_DIM), lambda i: (0, 0)),
            pl.BlockSpec((1, CODE_DIM), lambda i: (0, 0)),
            pl.BlockSpec((CODE_DIM, K), lambda i: (0, 0)),
            pl.BlockSpec((K, CODE_DIM), lambda i: (0, 0)),
            pl.BlockSpec((CODE_DIM, D), lambda i: (0, 0)),
            pl.BlockSpec((1, D), lambda i: (0, 0)),
        ],
        out_specs=[
            pl.BlockSpec((TILE_M, D), lambda i: (i, 0)),
            pl.BlockSpec((TILE_M, 1), lambda i: (i, 0)),
            pl.BlockSpec((1, 1), lambda i: (0, 0)),
            pl.BlockSpec((1, 1), lambda i: (0, 0)),
        ],
        out_shape=[
            jax.ShapeDtypeStruct((N, D), jnp.float32),
            jax.ShapeDtypeStruct((N, 1), jnp.int32),
            jax.ShapeDtypeStruct((1, 1), jnp.float32),
            jax.ShapeDtypeStruct((1, 1), jnp.float32),
        ],
        scratch_shapes=[
            pltpu.VMEM((8, K), jnp.float32),
            pltpu.VMEM((8, 128), jnp.float32),
        ],
    )(wf, lf, mf, wew, wel, wem, b_enc.reshape(1, CODE_DIM), cbt, cbb,
      wdb, b_dec.reshape(1, D))

    return (recon.reshape(B, T, D), codes.reshape(B, T),
            commit[0, 0], None, ent[0, 0])


# speedup vs baseline: 1.6693x; 1.6693x over previous
import jax
import jax.numpy as jnp
from jax.experimental import pallas as pl
from jax.experimental.pallas import tpu as pltpu

B, T = 16, 750
DW, DL, DM = 1280, 1024, 1024
D = DW + DL + DM
CODE_DIM, K = 256, 1024
N = B * T


def _body(w_ref, l_ref, m_ref, wew_ref, wel_ref, wem_ref, be_ref, cbt_ref,
          cbb_ref, wd_ref, bd_ref,
          recon_ref, codes_ref, commit_ref, ent_ref,
          acc_ref, csum_ref):
    i = pl.program_id(0)

    @pl.when(i == 0)
    def _init():
        acc_ref[...] = jnp.zeros_like(acc_ref)
        cbt = cbt_ref[...]
        acc_ref[1:2, :] = jnp.sum(cbt * cbt, axis=0, keepdims=True)
        csum_ref[...] = jnp.zeros_like(csum_ref)

    ze = (jnp.dot(w_ref[0], wew_ref[...], preferred_element_type=jnp.float32)
          + jnp.dot(l_ref[0], wel_ref[...], preferred_element_type=jnp.float32)
          + jnp.dot(m_ref[0], wem_ref[...], preferred_element_type=jnp.float32)
          + be_ref[...])

    z2 = jnp.sum(ze * ze, axis=1, keepdims=True)
    zc = jnp.dot(ze, cbt_ref[...], preferred_element_type=jnp.float32)
    d = z2 - 2.0 * zc + acc_ref[1:2, :]

    dmin = jnp.min(d, axis=1, keepdims=True)
    kiota = jax.lax.broadcasted_iota(jnp.int32, d.shape, 1)
    codes = jnp.min(jnp.where(d == dmin, kiota, K), axis=1,
                    keepdims=True)
    codes_ref[0] = codes

    p = jnp.exp(dmin - d)
    s = jnp.sum(p, axis=1, keepdims=True)
    acc_ref[0:1, :] = acc_ref[0:1, :] + jnp.sum(p / s, axis=0, keepdims=True)
    csum_ref[0:1, 0:1] = csum_ref[0:1, 0:1] + jnp.sum(dmin, axis=0,
                                                      keepdims=True)

    onehot = (kiota == codes).astype(jnp.bfloat16)
    zq = jnp.dot(onehot, cbb_ref[...],
                 preferred_element_type=jnp.float32)
    zq_st = ze + (zq - ze)
    recon_ref[0] = (jnp.dot(zq_st.astype(jnp.bfloat16), wd_ref[...],
                            preferred_element_type=jnp.float32)
                    + bd_ref[...])

    @pl.when(i == B - 1)
    def _fin():
        commit_ref[...] = csum_ref[0:1, 0:1] / (N * CODE_DIM)
        avg = acc_ref[0:1, :] / N
        ent_ref[...] = jnp.sum(avg * jnp.log(avg + 1e-10), axis=1,
                               keepdims=True)


@jax.jit
def kernel(whisper_feat, wavlm_feat, muq_feat, W_enc, b_enc, codebook,
           W_dec, b_dec):
    wew = W_enc[:DW]
    wel = W_enc[DW:DW + DL]
    wem = W_enc[DW + DL:]
    cbt = codebook.T
    cbb = codebook.astype(jnp.bfloat16)
    wdb = W_dec.astype(jnp.bfloat16)

    recon, codes, commit, ent = pl.pallas_call(
        _body,
        grid=(B,),
        in_specs=[
            pl.BlockSpec((1, T, DW), lambda i: (i, 0, 0)),
            pl.BlockSpec((1, T, DL), lambda i: (i, 0, 0)),
            pl.BlockSpec((1, T, DM), lambda i: (i, 0, 0)),
            pl.BlockSpec((DW, CODE_DIM), lambda i: (0, 0)),
            pl.BlockSpec((DL, CODE_DIM), lambda i: (0, 0)),
            pl.BlockSpec((DM, CODE_DIM), lambda i: (0, 0)),
            pl.BlockSpec((1, CODE_DIM), lambda i: (0, 0)),
            pl.BlockSpec((CODE_DIM, K), lambda i: (0, 0)),
            pl.BlockSpec((K, CODE_DIM), lambda i: (0, 0)),
            pl.BlockSpec((CODE_DIM, D), lambda i: (0, 0)),
            pl.BlockSpec((1, D), lambda i: (0, 0)),
        ],
        out_specs=[
            pl.BlockSpec((1, T, D), lambda i: (i, 0, 0)),
            pl.BlockSpec((1, T, 1), lambda i: (i, 0, 0)),
            pl.BlockSpec((1, 1), lambda i: (0, 0)),
            pl.BlockSpec((1, 1), lambda i: (0, 0)),
        ],
        out_shape=[
            jax.ShapeDtypeStruct((B, T, D), jnp.float32),
            jax.ShapeDtypeStruct((B, T, 1), jnp.int32),
            jax.ShapeDtypeStruct((1, 1), jnp.float32),
            jax.ShapeDtypeStruct((1, 1), jnp.float32),
        ],
        scratch_shapes=[
            pltpu.VMEM((8, K), jnp.float32),
            pltpu.VMEM((8, 128), jnp.float32),
        ],
    )(whisper_feat, wavlm_feat, muq_feat, wew, wel, wem,
      b_enc.reshape(1, CODE_DIM), cbt, cbb, wdb, b_dec.reshape(1, D))

    return (recon, codes.reshape(B, T),
            commit[0, 0], None, ent[0, 0])
